# baseline (speedup 1.0000x reference)
import jax
import jax.numpy as jnp
from jax import lax
from jax.experimental import pallas as pl
from jax.experimental.pallas import tpu as pltpu
from jax.experimental.pallas import tpu_sc as plsc

VOCAB = 1000000
D = 32
B, T = 4096, 200
NW = 32
G = 128
NBT = B // G
UNITS = T * NBT
PER_W = UNITS // NW
KU = 4
N_CHUNKS = PER_W // KU
DT = D // 8

_LANES = 16


def _tanh16(x):
    t = jnp.exp(x * -2.0)
    return 2.0 / (1.0 + t) - 1.0


def _body(table_hbm, idx_hbm, out_hbm, idx_v, g_v, y_v,
          g0, g1, g2, s0, s1, s2):
    gs = (g0, g1, g2)
    ss = (s0, s1, s2)
    wid = lax.axis_index("s") * 2 + lax.axis_index("c")
    w_r0 = wid * PER_W

    def unit_tb(c, k):
        r = w_r0 + c * KU + k
        return r // NBT, lax.rem(r, NBT)

    def load_chunk(c, s):
        pltpu.sync_copy(idx_hbm.at[pl.ds(w_r0 + c * KU, KU)], idx_v.at[s])
        for k in range(KU):
            pltpu.async_copy(
                table_hbm.at[idx_v.at[s, k]],
                g_v.at[s, pl.ds(k * G, G)],
                gs[s],
            )

    def wait_gathers(s):
        pltpu.make_async_copy(
            table_hbm.at[pl.ds(0, KU * G)], g_v.at[s], gs[s],
        ).wait()

    def store_chunk(c, s):
        for k in range(KU):
            t, bt = unit_tb(c, k)
            pltpu.async_copy(y_v.at[s, k], out_hbm.at[t, :, bt], ss[s])

    def wait_stores(c, s):
        for k in range(KU):
            t, bt = unit_tb(c, k)
            pltpu.make_async_copy(
                y_v.at[s, k], out_hbm.at[t, :, bt], ss[s],
            ).wait()

    def compute(s):
        iota = lax.iota(jnp.int32, _LANES)
        bases = [iota + j * _LANES for j in range(G // _LANES)]
        g_slot = g_v.at[s]

        def col_step(i, _):
            k = i // D
            d = lax.rem(i, D)
            dt = d // 8
            di = lax.rem(d, 8)
            colv = jnp.full((_LANES,), 0, jnp.int32) + d
            koff = k * G
            for j in range(G // _LANES):
                x = plsc.load_gather(g_slot, [bases[j] + koff, colv])
                y_v[s, k, dt, di, pl.ds(j * _LANES, _LANES)] = _tanh16(x)
            return 0

        lax.fori_loop(0, KU * D, col_step, 0)

    def substep(c, s, sn):
        @pl.when(c >= 2)
        def _():
            wait_stores(c - 2, sn)

        load_chunk(c + 1, sn)
        wait_gathers(s)
        compute(s)
        store_chunk(c, s)

    load_chunk(0, 0)

    def trip(q, _):
        c0 = q * 3
        substep(c0, 0, 1)
        substep(c0 + 1, 1, 2)
        substep(c0 + 2, 2, 0)
        return 0

    lax.fori_loop(0, (N_CHUNKS - 2) // 3, trip, 0)
    substep(N_CHUNKS - 2, 0, 1)
    wait_stores(N_CHUNKS - 3, 2)
    wait_gathers(1)
    compute(1)
    store_chunk(N_CHUNKS - 1, 1)
    wait_stores(N_CHUNKS - 2, 0)
    wait_stores(N_CHUNKS - 1, 1)


@jax.jit
def kernel(input_ids, table):
    idxq = input_ids.astype(jnp.int32).T.reshape(UNITS, G)
    mesh = plsc.VectorSubcoreMesh(core_axis_name="c", subcore_axis_name="s")
    yq = pl.kernel(
        _body,
        out_type=jax.ShapeDtypeStruct((T, DT, NBT, 8, G), jnp.float32),
        mesh=mesh,
        compiler_params=pltpu.CompilerParams(
            use_tc_tiling_on_sc=False, needs_layout_passes=False),
        scratch_types=[
            pltpu.VMEM((3, KU, G), jnp.int32),
            pltpu.VMEM((3, KU * G, D), jnp.float32),
            pltpu.VMEM((3, KU, DT, 8, G), jnp.float32),
        ] + [pltpu.SemaphoreType.DMA] * 6,
    )(table, idxq)
    out = yq.transpose(0, 1, 3, 2, 4).reshape(T, D, B).transpose(2, 0, 1)
    return out


# speedup vs baseline: 1.2708x; 1.2708x over previous
import jax
import jax.numpy as jnp
from jax import lax
from jax.experimental import pallas as pl
from jax.experimental.pallas import tpu as pltpu
from jax.experimental.pallas import tpu_sc as plsc

VOCAB = 1000000
D = 32
B, T = 4096, 200
NW = 32
G = 128
NBT = B // G
UNITS = T * NBT
PER_W = UNITS // NW
KU = 4
N_CHUNKS = PER_W // KU
DT = D // 8

_LANES = 16
_RUNROLL = 4
_PADW = G + 1


def _tanh16(x):
    t = jnp.exp(x * -2.0)
    return 2.0 / (1.0 + t) - 1.0


def _body(table_hbm, idx_hbm, out_hbm, idx_v, g_v, y_v,
          g0, g1, g2, s0, s1, s2):
    gs = (g0, g1, g2)
    ss = (s0, s1, s2)
    wid = lax.axis_index("s") * 2 + lax.axis_index("c")
    w_r0 = wid * PER_W

    def unit_tb(c, k):
        r = w_r0 + c * KU + k
        return r // NBT, lax.rem(r, NBT)

    def load_chunk(c, s):
        pltpu.sync_copy(idx_hbm.at[pl.ds(w_r0 + c * KU, KU)], idx_v.at[s])
        for k in range(KU):
            pltpu.async_copy(
                table_hbm.at[idx_v.at[s, k]],
                g_v.at[s, pl.ds(k * G, G)],
                gs[s],
            )

    def wait_gathers(s):
        pltpu.make_async_copy(
            table_hbm.at[pl.ds(0, KU * G)], g_v.at[s], gs[s],
        ).wait()

    def store_chunk(c, s):
        for k in range(KU):
            t, bt = unit_tb(c, k)
            pltpu.async_copy(y_v.at[s, k, :, :, pl.ds(0, G)],
                             out_hbm.at[t, :, bt], ss[s])

    def wait_stores(c, s):
        for k in range(KU):
            t, bt = unit_tb(c, k)
            pltpu.make_async_copy(
                y_v.at[s, k, :, :, pl.ds(0, G)], out_hbm.at[t, :, bt], ss[s],
            ).wait()

    def compute(s):
        iota = lax.iota(jnp.int32, _LANES)
        dtv = iota // 8
        div = lax.rem(iota, 8)
        g_slot = g_v.at[s]

        def row_step(i, _):
            r0 = i * _RUNROLL
            for u in range(_RUNROLL):
                r = r0 + u
                k = r // G
                colv = jnp.full((_LANES,), 0, jnp.int32) + lax.rem(r, G)
                y_unit = y_v.at[s].at[k]
                for h in range(D // _LANES):
                    x = g_slot[r, pl.ds(h * _LANES, _LANES)]
                    plsc.store_scatter(
                        y_unit, [dtv + 2 * h, div, colv], _tanh16(x))
            return 0

        lax.fori_loop(0, KU * G // _RUNROLL, row_step, 0)

    def substep(c, s, sn):
        @pl.when(c >= 2)
        def _():
            wait_stores(c - 2, sn)

        load_chunk(c + 1, sn)
        wait_gathers(s)
        compute(s)
        store_chunk(c, s)

    load_chunk(0, 0)

    def trip(q, _):
        c0 = q * 3
        substep(c0, 0, 1)
        substep(c0 + 1, 1, 2)
        substep(c0 + 2, 2, 0)
        return 0

    lax.fori_loop(0, (N_CHUNKS - 2) // 3, trip, 0)
    substep(N_CHUNKS - 2, 0, 1)
    wait_stores(N_CHUNKS - 3, 2)
    wait_gathers(1)
    compute(1)
    store_chunk(N_CHUNKS - 1, 1)
    wait_stores(N_CHUNKS - 2, 0)
    wait_stores(N_CHUNKS - 1, 1)


@jax.jit
def kernel(input_ids, table):
    idxq = input_ids.astype(jnp.int32).T.reshape(UNITS, G)
    mesh = plsc.VectorSubcoreMesh(core_axis_name="c", subcore_axis_name="s")
    yq = pl.kernel(
        _body,
        out_type=jax.ShapeDtypeStruct((T, DT, NBT, 8, G), jnp.float32),
        mesh=mesh,
        compiler_params=pltpu.CompilerParams(
            use_tc_tiling_on_sc=False, needs_layout_passes=False),
        scratch_types=[
            pltpu.VMEM((3, KU, G), jnp.int32),
            pltpu.VMEM((3, KU * G, D), jnp.float32),
            pltpu.VMEM((3, KU, DT, 8, _PADW), jnp.float32),
        ] + [pltpu.SemaphoreType.DMA] * 6,
    )(table, idxq)
    out = yq.transpose(0, 1, 3, 2, 4).reshape(T, D, B).transpose(2, 0, 1)
    return out
